# Initial kernel scaffold; baseline (speedup 1.0000x reference)
import jax
import jax.numpy as jnp
from jax.experimental import pallas as pl


def kernel(x, W1, asrc1, adst1, b1, W2, asrc2, adst2, b2, W3, asrc3, adst3, b3, W4, asrc4, adst4, b4, fc1_w, fc1_b, fc2_w, fc2_b, edge_index, batch):
    raise NotImplementedError("write your pallas kernel here")


# speedup vs baseline: 43.5487x; 43.5487x over previous
import functools

import jax
import jax.numpy as jnp
from jax import lax
from jax.experimental import pallas as pl
from jax.experimental.pallas import tpu as pltpu
from jax.experimental.pallas import tpu_sc as plsc

N_NODES = 10000
HEADS = 8
OUT = 16
HO = 128
G_GRP = 64
NP = 10240
RPW = NP // 16
K = 128
NC = 2
NS = 16
NW = NC * NS


def _splat(v, lane):
    dnums = lax.GatherDimensionNumbers(offset_dims=(), collapsed_slice_dims=(0,),
                                       start_index_map=(0,))
    idx = jnp.full((16, 1), lane, jnp.int32)
    return lax.gather(v, idx, dnums, (1,),
                      mode=lax.GatherScatterMode.PROMISE_IN_BOUNDS)


def _mesh():
    return plsc.VectorSubcoreMesh(core_axis_name="c", subcore_axis_name="s")


def _tc_layer(o0, o1, b_row, wt, amat, *, first):
    BM = 512

    def body(o0_ref, o1_ref, b_ref, wt_ref, amat_ref, hw_ref, es_ref, ed_ref):
        i = pl.program_id(0)
        if first:
            h = o0_ref[...]
        else:
            h = o0_ref[...] + o1_ref[...] + b_ref[...]
            h = jnp.where(h > 0, h, jnp.exp(h) - 1.0)
        rows = i * BM + lax.broadcasted_iota(jnp.int32, (BM, 1), 0)
        mask = rows < N_NODES
        h = jnp.where(mask, h, 0.0)
        hw = jnp.dot(h, wt_ref[...], preferred_element_type=jnp.float32)
        hw_ref[...] = hw
        esd = jnp.dot(hw, amat_ref[...], preferred_element_type=jnp.float32)
        esd = jnp.where(mask, esd, -1e30)
        es_ref[...] = esd[:, :16]
        ed_ref[...] = esd[:, 16:]

    return pl.pallas_call(
        body,
        grid=(NP // BM,),
        in_specs=[
            pl.BlockSpec((BM, HO), lambda i: (i, 0)),
            pl.BlockSpec((BM, HO), lambda i: (i, 0)),
            pl.BlockSpec((1, HO), lambda i: (0, 0)),
            pl.BlockSpec((HO, HO), lambda i: (0, 0)),
            pl.BlockSpec((HO, 32), lambda i: (0, 0)),
        ],
        out_specs=[
            pl.BlockSpec((BM, HO), lambda i: (i, 0)),
            pl.BlockSpec((BM, 16), lambda i: (i, 0)),
            pl.BlockSpec((BM, 16), lambda i: (i, 0)),
        ],
        out_shape=[
            jax.ShapeDtypeStruct((NP, HO), jnp.float32),
            jax.ShapeDtypeStruct((NP, 16), jnp.float32),
            jax.ShapeDtypeStruct((NP, 16), jnp.float32),
        ],
    )(o0, o1, b_row, wt, amat)


def _sc_pass1(src, dst, es16, ed16, z16, *, nblk):
    ep = nblk * K * NW

    @functools.partial(
        pl.kernel,
        out_type=[
            jax.ShapeDtypeStruct((ep, 16), jnp.float32),
            jax.ShapeDtypeStruct((NC * NP, 16), jnp.float32),
        ],
        mesh=_mesh(),
        compiler_params=pltpu.CompilerParams(use_tc_tiling_on_sc=False),
        scratch_types=[
            pltpu.VMEM((K,), jnp.int32),
            pltpu.VMEM((K,), jnp.int32),
            pltpu.VMEM((K, 16), jnp.float32),
            pltpu.VMEM((K, 16), jnp.float32),
            pltpu.VMEM((K, 16), jnp.float32),
            pltpu.VMEM_SHARED((NP, 16), jnp.float32),
            pltpu.SemaphoreType.DMA,
        ],
    )
    def run(src_h, dst_h, es_h, ed_h, z16_h, exd_o, den_o,
            idx_s, idx_d, bufs, bufd, exb, den_sh, sem):
        cid = lax.axis_index("c")
        sid = lax.axis_index("s")
        w = cid * NS + sid
        pltpu.sync_copy(z16_h, den_sh.at[pl.ds(sid * RPW, RPW)])
        plsc.subcore_barrier()

        @pl.loop(0, nblk)
        def _blk(blk):
            base = (w * nblk + blk) * K
            pltpu.sync_copy(src_h.at[pl.ds(base, K)], idx_s)
            pltpu.sync_copy(dst_h.at[pl.ds(base, K)], idx_d)
            pltpu.async_copy(es_h.at[idx_s], bufs, sem).wait()
            pltpu.async_copy(ed_h.at[idx_d], bufd, sem).wait()

            @pl.loop(0, K)
            def _edge(j):
                e = bufs[j] + bufd[j]
                e = jnp.where(e > 0, e, 0.2 * e)
                exb[j] = jnp.exp(e)

            pltpu.sync_copy(exb, exd_o.at[pl.ds(base, K)])
            pltpu.sync_copy(exb, den_sh.at[idx_d], add=True)

        plsc.subcore_barrier()
        pltpu.sync_copy(den_sh.at[pl.ds(sid * RPW, RPW)],
                        den_o.at[pl.ds(cid * NP + sid * RPW, RPW)])

    return run(src, dst, es16, ed16, z16)


def _sc_pass2(src, dst, exd, den, hw, z128, *, nblk):
    @functools.partial(
        pl.kernel,
        out_type=[jax.ShapeDtypeStruct((NC * NP, HO), jnp.float32)],
        mesh=_mesh(),
        compiler_params=pltpu.CompilerParams(use_tc_tiling_on_sc=False),
        scratch_types=[
            pltpu.VMEM((K,), jnp.int32),
            pltpu.VMEM((K,), jnp.int32),
            pltpu.VMEM((K,), jnp.int32),
            pltpu.VMEM((K, HO), jnp.float32),
            pltpu.VMEM((K, 16), jnp.float32),
            pltpu.VMEM((K, 16), jnp.float32),
            pltpu.VMEM((K, 16), jnp.float32),
            pltpu.VMEM_SHARED((NP, HO), jnp.float32),
            pltpu.SemaphoreType.DMA,
        ],
    )
    def run(src_h, dst_h, exd_h, den_h, hw_h, z128_h, o_o,
            idx_s, idx_d, idx_d2, hbuf, d0, d1, fbuf, o_sh, sem):
        cid = lax.axis_index("c")
        sid = lax.axis_index("s")
        w = cid * NS + sid
        pltpu.sync_copy(z128_h, o_sh.at[pl.ds(sid * RPW, RPW)])
        plsc.subcore_barrier()

        @pl.loop(0, nblk)
        def _blk(blk):
            base = (w * nblk + blk) * K
            pltpu.sync_copy(src_h.at[pl.ds(base, K)], idx_s)
            pltpu.sync_copy(dst_h.at[pl.ds(base, K)], idx_d)

            @pl.loop(0, K // 16)
            def _t(t):
                idx_d2[pl.ds(t * 16, 16)] = idx_d[pl.ds(t * 16, 16)] + NP

            pltpu.async_copy(hw_h.at[idx_s], hbuf, sem).wait()
            pltpu.async_copy(den_h.at[idx_d], d0, sem).wait()
            pltpu.async_copy(den_h.at[idx_d2], d1, sem).wait()
            pltpu.sync_copy(exd_h.at[pl.ds(base, K)], fbuf)

            @pl.loop(0, K)
            def _edge(j):
                f = fbuf[j] / (d0[j] + d1[j] + 1e-16)
                fbuf[j] = f

            @pl.loop(0, K)
            def _edge2(j):
                frow = fbuf[j]
                for v in range(HEADS):
                    sp = _splat(frow, v)
                    hbuf[j, pl.ds(v * 16, 16)] = hbuf[j, pl.ds(v * 16, 16)] * sp

            pltpu.sync_copy(hbuf, o_sh.at[idx_d], add=True)

        plsc.subcore_barrier()
        pltpu.sync_copy(o_sh.at[pl.ds(sid * RPW, RPW)],
                        o_o.at[pl.ds(cid * NP + sid * RPW, RPW)])

    return run(src, dst, exd, den, hw, z128)


def _tc_final(o0, o1, b_row, batch_row, batch_col, fc1_wt, fc1_b_row, fc2_wt,
              fc2_b_row):
    def body(o0_ref, o1_ref, b_ref, brow_ref, bcol_ref, w1_ref, b1_ref,
             w2_ref, b2_ref, out_ref, mx_ref):
        h = o0_ref[...] + o1_ref[...] + b_ref[...]
        h = jnp.where(h > 0, h, jnp.exp(h) - 1.0)
        bcol = bcol_ref[...]
        gids = lax.broadcasted_iota(jnp.int32, (G_GRP, NP), 0)
        onehot = (brow_ref[...] == gids).astype(jnp.float32)
        sums = jnp.dot(onehot, h, preferred_element_type=jnp.float32)
        cnt = jnp.sum(onehot, axis=1, keepdims=True)
        mean = sums / jnp.maximum(cnt, 1.0)
        for g in range(G_GRP):
            m = jnp.where(bcol == g, h, -2.0)
            mx_ref[g:g + 1, :] = jnp.max(m, axis=0, keepdims=True)
        mx = mx_ref[...]
        mx = jnp.where(mx <= -1.5, 0.0, mx)
        z = jnp.concatenate([mean, mx], axis=1)
        z1 = jnp.dot(z, w1_ref[...], preferred_element_type=jnp.float32)
        z1 = jnp.maximum(z1 + b1_ref[...], 0.0)
        logits = jnp.dot(z1, w2_ref[...], preferred_element_type=jnp.float32)
        logits = logits + b2_ref[...]
        l0 = logits[:, 0:1]
        l1 = logits[:, 1:2]
        m2 = jnp.maximum(l0, l1)
        ls = m2 + jnp.log(jnp.exp(l0 - m2) + jnp.exp(l1 - m2))
        out_ref[...] = jnp.concatenate([l0 - ls, l1 - ls], axis=1)

    return pl.pallas_call(
        body,
        out_shape=jax.ShapeDtypeStruct((G_GRP, 2), jnp.float32),
        scratch_shapes=[pltpu.VMEM((G_GRP, HO), jnp.float32)],
    )(o0, o1, b_row, batch_row, batch_col, fc1_wt, fc1_b_row, fc2_wt,
      fc2_b_row)


def _amat(a_s, a_d):
    c = jnp.arange(HO, dtype=jnp.int32)
    bd_s = jnp.zeros((HO, HEADS), jnp.float32).at[c, c // OUT].set(a_s.reshape(-1))
    bd_d = jnp.zeros((HO, HEADS), jnp.float32).at[c, c // OUT].set(a_d.reshape(-1))
    return jnp.concatenate([bd_s, bd_s, bd_d, bd_d], axis=1)


def kernel(x, W1, asrc1, adst1, b1, W2, asrc2, adst2, b2, W3, asrc3, adst3,
           b3, W4, asrc4, adst4, b4, fc1_w, fc1_b, fc2_w, fc2_b, edge_index,
           batch):
    e_raw = edge_index.shape[1]
    n_tot = e_raw + N_NODES
    nblk = -(-n_tot // (NW * K))
    ep = nblk * K * NW

    loop = jnp.arange(N_NODES, dtype=jnp.int32)
    padi = jnp.full((ep - n_tot,), N_NODES, jnp.int32)
    src = jnp.concatenate([edge_index[0].astype(jnp.int32), loop, padi])
    dst = jnp.concatenate([edge_index[1].astype(jnp.int32), loop, padi])

    x_pad = jnp.pad(x, ((0, NP - N_NODES), (0, 0)))
    z16 = jnp.zeros((RPW, 16), jnp.float32)
    z128 = jnp.zeros((RPW, HO), jnp.float32)
    zrow = jnp.zeros((1, HO), jnp.float32)

    layers = [(W1, asrc1, adst1, b1), (W2, asrc2, adst2, b2),
              (W3, asrc3, adst3, b3), (W4, asrc4, adst4, b4)]
    wts = [w.T for (w, _, _, _) in layers]
    amats = [_amat(a_s, a_d) for (_, a_s, a_d, _) in layers]
    brows = [b.reshape(1, HO) for (_, _, _, b) in layers]

    hw, es16, ed16 = _tc_layer(x_pad, x_pad, zrow, wts[0], amats[0],
                               first=True)
    for li in range(4):
        exd, den = _sc_pass1(src, dst, es16, ed16, z16, nblk=nblk)
        (o2,) = _sc_pass2(src, dst, exd, den, hw, z128, nblk=nblk)
        o0, o1 = o2[:NP], o2[NP:]
        if li < 3:
            hw, es16, ed16 = _tc_layer(o0, o1, brows[li], wts[li + 1],
                                       amats[li + 1], first=False)

    batch_row = jnp.pad(batch.astype(jnp.int32), (0, NP - N_NODES),
                        constant_values=G_GRP).reshape(1, NP)
    batch_col = batch_row.reshape(NP, 1)
    fc1_wt = jnp.pad(fc1_w, ((0, 28), (0, 0))).T
    fc1_b_row = jnp.pad(fc1_b, (0, 28)).reshape(1, HO)
    fc2_wt = jnp.pad(fc2_w, ((0, 126), (0, 28))).T
    fc2_b_row = jnp.pad(fc2_b, (0, 126)).reshape(1, HO)
    return _tc_final(o0, o1, brows[3], batch_row, batch_col, fc1_wt,
                     fc1_b_row, fc2_wt, fc2_b_row)
